# baseline (speedup 1.0000x reference)
import jax
import jax.numpy as jnp
from jax.experimental import pallas as pl
from jax.experimental.pallas import tpu as pltpu

_TM = 256


def _encode_kernel(x_ref, w_ref, b_ref, pos_ref, o_ref):
    x = x_ref[0]
    acc = jax.lax.dot_general(
        x, w_ref[...], (((1,), (1,)), ((), ())),
        preferred_element_type=jnp.float32,
    )
    o_ref[0] = acc + b_ref[...] + pos_ref[...]


def kernel(patch, W, b, pos_table):
    B, N, D = patch.shape
    P = W.shape[0]
    b2 = b.reshape(1, P)
    nt = N // _TM
    return pl.pallas_call(
        _encode_kernel,
        grid=(B, nt),
        in_specs=[
            pl.BlockSpec((1, _TM, D), lambda i, j: (i, j, 0)),
            pl.BlockSpec((P, D), lambda i, j: (0, 0)),
            pl.BlockSpec((1, P), lambda i, j: (0, 0)),
            pl.BlockSpec((_TM, P), lambda i, j: (j, 0)),
        ],
        out_specs=pl.BlockSpec((1, _TM, P), lambda i, j: (i, j, 0)),
        out_shape=jax.ShapeDtypeStruct((B, N, P), jnp.float32),
        compiler_params=pltpu.CompilerParams(
            dimension_semantics=("parallel", "arbitrary"),
        ),
    )(patch, W, b2, pos_table)


# speedup vs baseline: 2.2996x; 2.2996x over previous
import jax
import jax.numpy as jnp
from jax.experimental import pallas as pl
from jax.experimental.pallas import tpu as pltpu

_BB = 2


def _encode_kernel(x_ref, w_ref, b_ref, pos_ref, o_ref):
    bb, n, d = x_ref.shape
    x = x_ref[...].reshape(bb * n, d)
    acc = jax.lax.dot_general(
        x, w_ref[...], (((1,), (1,)), ((), ())),
        preferred_element_type=jnp.float32,
    )
    p = acc.shape[1]
    o_ref[...] = acc.reshape(bb, n, p) + b_ref[...] + pos_ref[...][None]


def kernel(patch, W, b, pos_table):
    B, N, D = patch.shape
    P = W.shape[0]
    b2 = b.reshape(1, P)
    return pl.pallas_call(
        _encode_kernel,
        grid=(B // _BB,),
        in_specs=[
            pl.BlockSpec((_BB, N, D), lambda i: (i, 0, 0)),
            pl.BlockSpec((P, D), lambda i: (0, 0)),
            pl.BlockSpec((1, P), lambda i: (0, 0)),
            pl.BlockSpec((N, P), lambda i: (0, 0)),
        ],
        out_specs=pl.BlockSpec((_BB, N, P), lambda i: (i, 0, 0)),
        out_shape=jax.ShapeDtypeStruct((B, N, P), jnp.float32),
        compiler_params=pltpu.CompilerParams(
            dimension_semantics=("parallel",),
        ),
    )(patch, W, b2, pos_table)


# speedup vs baseline: 2.3512x; 1.0224x over previous
import jax
import jax.numpy as jnp
from jax.experimental import pallas as pl
from jax.experimental.pallas import tpu as pltpu

_BB = 4


def _encode_kernel(x_ref, w_ref, b_ref, pos_ref, o_ref):
    bb, n, d = x_ref.shape
    x = x_ref[...].reshape(bb * n, d)
    acc = jax.lax.dot_general(
        x, w_ref[...], (((1,), (1,)), ((), ())),
        preferred_element_type=jnp.float32,
    )
    p = acc.shape[1]
    o_ref[...] = acc.reshape(bb, n, p) + b_ref[...] + pos_ref[...][None]


def kernel(patch, W, b, pos_table):
    B, N, D = patch.shape
    P = W.shape[0]
    b2 = b.reshape(1, P)
    return pl.pallas_call(
        _encode_kernel,
        grid=(B // _BB,),
        in_specs=[
            pl.BlockSpec((_BB, N, D), lambda i: (i, 0, 0)),
            pl.BlockSpec((P, D), lambda i: (0, 0)),
            pl.BlockSpec((1, P), lambda i: (0, 0)),
            pl.BlockSpec((N, P), lambda i: (0, 0)),
        ],
        out_specs=pl.BlockSpec((_BB, N, P), lambda i: (i, 0, 0)),
        out_shape=jax.ShapeDtypeStruct((B, N, P), jnp.float32),
        compiler_params=pltpu.CompilerParams(
            dimension_semantics=("parallel",),
        ),
    )(patch, W, b2, pos_table)


# speedup vs baseline: 2.3561x; 1.0021x over previous
import jax
import jax.numpy as jnp
from jax.experimental import pallas as pl
from jax.experimental.pallas import tpu as pltpu

_BB = 2
_NS = 2


def _encode_kernel(x1_ref, x2_ref, w_ref, b_ref, pos_ref, o_ref):
    for s, xr in enumerate((x1_ref, x2_ref)):
        bb, n, d = xr.shape
        x = xr[...].reshape(bb * n, d)
        acc = jax.lax.dot_general(
            x, w_ref[...], (((1,), (1,)), ((), ())),
            preferred_element_type=jnp.float32,
        )
        p = acc.shape[1]
        o_ref[s * bb:(s + 1) * bb] = (
            acc.reshape(bb, n, p) + b_ref[...] + pos_ref[...][None]
        )


def kernel(patch, W, b, pos_table):
    B, N, D = patch.shape
    P = W.shape[0]
    b2 = b.reshape(1, P)
    nsteps = B // (_BB * _NS)
    return pl.pallas_call(
        _encode_kernel,
        grid=(nsteps,),
        in_specs=[
            pl.BlockSpec((_BB, N, D), lambda i: (_NS * i, 0, 0)),
            pl.BlockSpec((_BB, N, D), lambda i: (_NS * i + 1, 0, 0)),
            pl.BlockSpec((P, D), lambda i: (0, 0)),
            pl.BlockSpec((1, P), lambda i: (0, 0)),
            pl.BlockSpec((N, P), lambda i: (0, 0)),
        ],
        out_specs=pl.BlockSpec((_BB * _NS, N, P), lambda i: (i, 0, 0)),
        out_shape=jax.ShapeDtypeStruct((B, N, P), jnp.float32),
        compiler_params=pltpu.CompilerParams(
            dimension_semantics=("parallel",),
        ),
    )(patch, patch, W, b2, pos_table)


# speedup vs baseline: 2.3576x; 1.0006x over previous
import jax
import jax.numpy as jnp
from jax.experimental import pallas as pl
from jax.experimental.pallas import tpu as pltpu

_NBUF = 4


def _encode_kernel(x_hbm, w_ref, b_ref, pos_ref, o_ref, xbuf, sems):
    i = pl.program_id(0)
    nsteps = pl.num_programs(0)

    @pl.when(i == 0)
    def _warmup():
        for k in range(_NBUF):
            pltpu.make_async_copy(x_hbm.at[k], xbuf.at[k], sems.at[k]).start()

    slot = jax.lax.rem(i, _NBUF)
    pltpu.make_async_copy(x_hbm.at[i], xbuf.at[slot], sems.at[slot]).wait()

    acc = jax.lax.dot_general(
        xbuf[slot], w_ref[...], (((1,), (1,)), ((), ())),
        preferred_element_type=jnp.float32,
    )
    o_ref[0] = acc + b_ref[...] + pos_ref[...]

    nxt = i + _NBUF
    nslot = jax.lax.rem(nxt, _NBUF)

    @pl.when(nxt < nsteps)
    def _prefetch():
        pltpu.make_async_copy(x_hbm.at[nxt], xbuf.at[nslot], sems.at[nslot]).start()


def kernel(patch, W, b, pos_table):
    B, N, D = patch.shape
    P = W.shape[0]
    b2 = b.reshape(1, P)
    return pl.pallas_call(
        _encode_kernel,
        grid=(B,),
        in_specs=[
            pl.BlockSpec(memory_space=pltpu.HBM),
            pl.BlockSpec((P, D), lambda i: (0, 0)),
            pl.BlockSpec((1, P), lambda i: (0, 0)),
            pl.BlockSpec((N, P), lambda i: (0, 0)),
        ],
        out_specs=pl.BlockSpec((1, N, P), lambda i: (i, 0, 0)),
        out_shape=jax.ShapeDtypeStruct((B, N, P), jnp.float32),
        scratch_shapes=[
            pltpu.VMEM((_NBUF, N, D), jnp.float32),
            pltpu.SemaphoreType.DMA((_NBUF,)),
        ],
        compiler_params=pltpu.CompilerParams(
            dimension_semantics=("arbitrary",),
        ),
    )(patch, W, b2, pos_table)
